# baseline (speedup 1.0000x reference)
import functools

import jax
import jax.numpy as jnp
from jax import lax
from jax.experimental import pallas as pl
from jax.experimental.pallas import tpu as pltpu
from jax.experimental.pallas import tpu_sc as plsc

N = 10000
E = 320000
D = 128
R = 8

NC = 2
NS = 16
LANE = 112
CH = 96
EPAD = NC * NS * CH * LANE
NPAD = 10112
BN = 1000


def _pre_body(x_ref, w_ref, lw_ref, hw_ref, loop_ref):
    x = x_ref[...]
    for r in range(R):
        hw_ref[:, r, :] = jnp.dot(x, w_ref[r], preferred_element_type=jnp.float32)
    loop_ref[...] = jnp.dot(x, lw_ref[...], preferred_element_type=jnp.float32)


_pre = pl.pallas_call(
    _pre_body,
    grid=(N // BN,),
    in_specs=[
        pl.BlockSpec((BN, D), lambda i: (i, 0)),
        pl.BlockSpec((R, D, D), lambda i: (0, 0, 0)),
        pl.BlockSpec((D, D), lambda i: (0, 0)),
    ],
    out_specs=[
        pl.BlockSpec((BN, R, D), lambda i: (i, 0, 0)),
        pl.BlockSpec((BN, D), lambda i: (i, 0)),
    ],
    out_shape=[
        jax.ShapeDtypeStruct((N, R, D), jnp.float32),
        jax.ShapeDtypeStruct((N, D), jnp.float32),
    ],
)


def _post_body(parts_ref, loop_ref, g_ref, b_ref, bias_ref, o_ref, *, act):
    aggv = parts_ref[0] + parts_ref[1]
    mean = jnp.mean(aggv, axis=-1, keepdims=True)
    xc = aggv - mean
    var = jnp.mean(xc * xc, axis=-1, keepdims=True)
    h = (xc * lax.rsqrt(var + 1e-5) * g_ref[...] + b_ref[...]
         + bias_ref[...] + loop_ref[...])
    o_ref[...] = jnp.maximum(h, 0.0) if act else h


def _make_post(act):
    return pl.pallas_call(
        functools.partial(_post_body, act=act),
        grid=(N // BN,),
        in_specs=[
            pl.BlockSpec((2, BN, D), lambda i: (0, i, 0)),
            pl.BlockSpec((BN, D), lambda i: (i, 0)),
            pl.BlockSpec((1, D), lambda i: (0, 0)),
            pl.BlockSpec((1, D), lambda i: (0, 0)),
            pl.BlockSpec((1, D), lambda i: (0, 0)),
        ],
        out_specs=pl.BlockSpec((BN, D), lambda i: (i, 0)),
        out_shape=jax.ShapeDtypeStruct((N, D), jnp.float32),
    )


_post_act = _make_post(True)
_post_noact = _make_post(False)


NBUF = 3
RING = 4
RINGD = 8
DLEAD = 5


def _sc_body(hw_hbm, idx_hbm, dst_hbm, zeros_hbm, out_hbm, *rest):
    pos = 0
    islots = rest[pos:pos + RING]; pos += RING
    isems = rest[pos:pos + RING]; pos += RING
    dslots = rest[pos:pos + RINGD]; pos += RINGD
    dsems = rest[pos:pos + RINGD]; pos += RINGD
    bufs = rest[pos:pos + NBUF]; pos += NBUF
    gsems = rest[pos:pos + NBUF]; pos += NBUF
    ssems = rest[pos:pos + NBUF]; pos += NBUF
    agg = rest[pos]
    c = lax.axis_index("c")
    s = lax.axis_index("s")
    zrows = NPAD // NS
    pltpu.sync_copy(zeros_hbm, agg.at[pl.ds(s * zrows, zrows)])

    def load_idx(j, i):
        pltpu.async_copy(idx_hbm.at[c, s, j], islots[i], isems[i])

    def wait_i(i):
        pltpu.make_async_copy(idx_hbm.at[c, s, 0], islots[i], isems[i]).wait()

    def load_dst(j, i):
        pltpu.async_copy(dst_hbm.at[c, s, j], dslots[i], dsems[i])

    def wait_d(i):
        pltpu.make_async_copy(dst_hbm.at[c, s, 0], dslots[i], dsems[i]).wait()

    def gather(i, b):
        pltpu.async_copy(hw_hbm.at[islots[i]], bufs[b], gsems[b])

    def wait_g(b):
        pltpu.make_async_copy(hw_hbm.at[islots[0]], bufs[b], gsems[b]).wait()

    def scatter(i, b):
        pltpu.async_copy(bufs[b], agg.at[dslots[i]], ssems[b], add=True)

    def wait_s(b):
        pltpu.make_async_copy(bufs[b], agg.at[dslots[0]], ssems[b]).wait()

    plsc.subcore_barrier()

    for i in range(RING):
        load_idx(i, i)
    for i in range(DLEAD):
        load_dst(i, i)
    wait_i(0)
    gather(0, 0)

    UN = NBUF * RINGD if RINGD % NBUF else RINGD
    assert UN % NBUF == 0 and UN % RING == 0 and UN % RINGD == 0
    assert CH % UN == 0 and DLEAD < RINGD

    def body(k, carry):
        for u in range(UN):
            j = k * UN + u

            wait_g(u % NBUF)

            @pl.when(j + RING < CH)
            def _():
                load_idx(j + RING, u % RING)

            wait_d(u % RINGD)
            scatter(u % RINGD, u % NBUF)

            @pl.when(j >= NBUF - 1)
            def _():
                wait_s((u + 1) % NBUF)

            @pl.when(j + DLEAD < CH)
            def _():
                load_dst(j + DLEAD, (u + DLEAD) % RINGD)

            @pl.when(j + 1 < CH)
            def _():
                wait_i((u + 1) % RING)
                gather((u + 1) % RING, (u + 1) % NBUF)

        return carry

    lax.fori_loop(0, CH // UN, body, 0)
    for k in range(1, NBUF):
        wait_s(k % NBUF)
    plsc.subcore_barrier()
    orows = NPAD // NS
    pltpu.sync_copy(agg.at[pl.ds(s * orows, orows)],
                    out_hbm.at[c, pl.ds(s * orows, orows)])


@functools.cache
def _get_sc_agg():
    return pl.kernel(
        _sc_body,
        out_type=jax.ShapeDtypeStruct((NC, NPAD, D), jnp.float32),
        mesh=plsc.VectorSubcoreMesh(core_axis_name="c", subcore_axis_name="s",
                                    num_cores=NC, num_subcores=NS),
        scratch_types=(
            [pltpu.VMEM((LANE,), jnp.int32) for _ in range(RING)]
            + [pltpu.SemaphoreType.DMA for _ in range(RING)]
            + [pltpu.VMEM((LANE,), jnp.int32) for _ in range(RINGD)]
            + [pltpu.SemaphoreType.DMA for _ in range(RINGD)]
            + [pltpu.VMEM((LANE, D), jnp.float32) for _ in range(NBUF)]
            + [pltpu.SemaphoreType.DMA for _ in range(2 * NBUF)]
            + [pltpu.VMEM_SHARED((NPAD, D), jnp.float32)]
        ),
    )


def kernel(feat, edge_index, etypes,
           W0, bias0, loopW0, ln_g0, ln_b0,
           W1, bias1, loopW1, ln_g1, ln_b1,
           W2, bias2, loopW2, ln_g2, ln_b2):
    src = edge_index[0]
    dst = edge_index[1]
    flat = src * R + etypes
    nw = NC * NS
    rl = E // nw
    padn = CH * LANE - rl
    karr = jnp.arange(padn, dtype=jnp.int32)
    warr = jnp.arange(nw, dtype=jnp.int32).reshape(nw, 1)
    jpt = (NPAD - N) // NS
    idx_pad = (karr * 331 + warr * 77) % (N * R)
    dst_pad = N + (warr % NS) * jpt + karr % jpt
    flat_p = jnp.concatenate([flat.reshape(nw, rl), idx_pad], axis=1)
    dst_p = jnp.concatenate([dst.reshape(nw, rl), dst_pad], axis=1)
    idx3 = flat_p.reshape(NC, NS, CH, LANE)
    dst3 = dst_p.reshape(NC, NS, CH, LANE)
    zeros = jnp.zeros((NPAD // NS, D), jnp.float32)

    h = feat
    layers = (
        (W0, bias0, loopW0, ln_g0, ln_b0, True),
        (W1, bias1, loopW1, ln_g1, ln_b1, True),
        (W2, bias2, loopW2, ln_g2, ln_b2, False),
    )
    for W, bias, loopW, g, b, act in layers:
        hw, loop = _pre(h, W, loopW)
        parts = _get_sc_agg()(hw.reshape(N * R, D), idx3, dst3, zeros)
        post = _post_act if act else _post_noact
        h = post(parts, loop, g.reshape(1, D), b.reshape(1, D),
                 bias.reshape(1, D))
    return h


# speedup vs baseline: 1.0944x; 1.0944x over previous
import functools

import jax
import jax.numpy as jnp
from jax import lax
from jax.experimental import pallas as pl
from jax.experimental.pallas import tpu as pltpu
from jax.experimental.pallas import tpu_sc as plsc

N = 10000
E = 320000
D = 128
R = 8

NC = 2
NS = 16
LANE = 128
CH = 80
EPAD = NC * NS * CH * LANE
NPAD = 10112
BN = 1000


def _pre_body(x_ref, w_ref, lw_ref, hw_ref, loop_ref):
    x = x_ref[...]
    for r in range(R):
        hw_ref[:, r, :] = jnp.dot(x, w_ref[r], preferred_element_type=jnp.float32)
    loop_ref[...] = jnp.dot(x, lw_ref[...], preferred_element_type=jnp.float32)


_pre = pl.pallas_call(
    _pre_body,
    grid=(N // BN,),
    in_specs=[
        pl.BlockSpec((BN, D), lambda i: (i, 0)),
        pl.BlockSpec((R, D, D), lambda i: (0, 0, 0)),
        pl.BlockSpec((D, D), lambda i: (0, 0)),
    ],
    out_specs=[
        pl.BlockSpec((BN, R, D), lambda i: (i, 0, 0)),
        pl.BlockSpec((BN, D), lambda i: (i, 0)),
    ],
    out_shape=[
        jax.ShapeDtypeStruct((N, R, D), jnp.float32),
        jax.ShapeDtypeStruct((N, D), jnp.float32),
    ],
)


def _post_body(parts_ref, loop_ref, g_ref, b_ref, bias_ref, o_ref, *, act):
    aggv = parts_ref[0] + parts_ref[1]
    mean = jnp.mean(aggv, axis=-1, keepdims=True)
    xc = aggv - mean
    var = jnp.mean(xc * xc, axis=-1, keepdims=True)
    h = (xc * lax.rsqrt(var + 1e-5) * g_ref[...] + b_ref[...]
         + bias_ref[...] + loop_ref[...])
    o_ref[...] = jnp.maximum(h, 0.0) if act else h


def _make_post(act):
    return pl.pallas_call(
        functools.partial(_post_body, act=act),
        grid=(N // BN,),
        in_specs=[
            pl.BlockSpec((2, BN, D), lambda i: (0, i, 0)),
            pl.BlockSpec((BN, D), lambda i: (i, 0)),
            pl.BlockSpec((1, D), lambda i: (0, 0)),
            pl.BlockSpec((1, D), lambda i: (0, 0)),
            pl.BlockSpec((1, D), lambda i: (0, 0)),
        ],
        out_specs=pl.BlockSpec((BN, D), lambda i: (i, 0)),
        out_shape=jax.ShapeDtypeStruct((N, D), jnp.float32),
    )


_post_act = _make_post(True)
_post_noact = _make_post(False)


def _ln_h(parts_ref, loop_ref, g_ref, b_ref, bias_ref):
    aggv = parts_ref[0] + parts_ref[1]
    mean = jnp.mean(aggv, axis=-1, keepdims=True)
    xc = aggv - mean
    var = jnp.mean(xc * xc, axis=-1, keepdims=True)
    h = (xc * lax.rsqrt(var + 1e-5) * g_ref[...] + b_ref[...]
         + bias_ref[...] + loop_ref[...])
    return jnp.maximum(h, 0.0)


def _bound_hw_body(parts_ref, loop_ref, g_ref, b_ref, bias_ref, w_ref,
                   hw_ref):
    h = _ln_h(parts_ref, loop_ref, g_ref, b_ref, bias_ref)
    for r in range(R):
        hw_ref[:, r, :] = jnp.dot(h, w_ref[r], preferred_element_type=jnp.float32)


def _bound_loop_body(parts_ref, loop_ref, g_ref, b_ref, bias_ref, lw_ref,
                     lo_ref):
    h = _ln_h(parts_ref, loop_ref, g_ref, b_ref, bias_ref)
    lo_ref[...] = jnp.dot(h, lw_ref[...], preferred_element_type=jnp.float32)


_BOUND_SPECS = [
    pl.BlockSpec((2, BN, D), lambda i: (0, i, 0)),
    pl.BlockSpec((BN, D), lambda i: (i, 0)),
    pl.BlockSpec((1, D), lambda i: (0, 0)),
    pl.BlockSpec((1, D), lambda i: (0, 0)),
    pl.BlockSpec((1, D), lambda i: (0, 0)),
]

_bound_hw = pl.pallas_call(
    _bound_hw_body,
    grid=(N // BN,),
    in_specs=_BOUND_SPECS + [pl.BlockSpec((R, D, D), lambda i: (0, 0, 0))],
    out_specs=pl.BlockSpec((BN, R, D), lambda i: (i, 0, 0)),
    out_shape=jax.ShapeDtypeStruct((N, R, D), jnp.float32),
)

_bound_loop = pl.pallas_call(
    _bound_loop_body,
    grid=(N // BN,),
    in_specs=_BOUND_SPECS + [pl.BlockSpec((D, D), lambda i: (0, 0))],
    out_specs=pl.BlockSpec((BN, D), lambda i: (i, 0)),
    out_shape=jax.ShapeDtypeStruct((N, D), jnp.float32),
)


NBUF = 2
RING = 4
RINGD = 8
DLEAD = 5


def _sc_body(hw_hbm, idx_hbm, dst_hbm, zeros_hbm, out_hbm, *rest):
    pos = 0
    islots = rest[pos:pos + RING]; pos += RING
    isems = rest[pos:pos + RING]; pos += RING
    dslots = rest[pos:pos + RINGD]; pos += RINGD
    dsems = rest[pos:pos + RINGD]; pos += RINGD
    bufs = rest[pos:pos + NBUF]; pos += NBUF
    gsems = rest[pos:pos + NBUF]; pos += NBUF
    ssems = rest[pos:pos + NBUF]; pos += NBUF
    agg = rest[pos]
    c = lax.axis_index("c")
    s = lax.axis_index("s")
    zrows = NPAD // NS
    pltpu.sync_copy(zeros_hbm, agg.at[pl.ds(s * zrows, zrows)])

    def load_idx(j, i):
        pltpu.async_copy(idx_hbm.at[c, s, j], islots[i], isems[i])

    def wait_i(i):
        pltpu.make_async_copy(idx_hbm.at[c, s, 0], islots[i], isems[i]).wait()

    def load_dst(j, i):
        pltpu.async_copy(dst_hbm.at[c, s, j], dslots[i], dsems[i])

    def wait_d(i):
        pltpu.make_async_copy(dst_hbm.at[c, s, 0], dslots[i], dsems[i]).wait()

    def gather(i, b):
        pltpu.async_copy(hw_hbm.at[islots[i]], bufs[b], gsems[b])

    def wait_g(b):
        pltpu.make_async_copy(hw_hbm.at[islots[0]], bufs[b], gsems[b]).wait()

    def scatter(i, b):
        pltpu.async_copy(bufs[b], agg.at[dslots[i]], ssems[b], add=True)

    def wait_s(b):
        pltpu.make_async_copy(bufs[b], agg.at[dslots[0]], ssems[b]).wait()

    plsc.subcore_barrier()

    for i in range(RING):
        load_idx(i, i)
    for i in range(DLEAD):
        load_dst(i, i)
    wait_i(0)
    gather(0, 0)

    UN = NBUF * RINGD if RINGD % NBUF else RINGD
    assert UN % NBUF == 0 and UN % RING == 0 and UN % RINGD == 0
    assert CH % UN == 0 and DLEAD < RINGD

    def body(k, carry):
        for u in range(UN):
            j = k * UN + u

            wait_g(u % NBUF)

            @pl.when(j + RING < CH)
            def _():
                load_idx(j + RING, u % RING)

            wait_d(u % RINGD)
            scatter(u % RINGD, u % NBUF)

            @pl.when(j >= NBUF - 1)
            def _():
                wait_s((u + 1) % NBUF)

            @pl.when(j + DLEAD < CH)
            def _():
                load_dst(j + DLEAD, (u + DLEAD) % RINGD)

            @pl.when(j + 1 < CH)
            def _():
                wait_i((u + 1) % RING)
                gather((u + 1) % RING, (u + 1) % NBUF)

        return carry

    lax.fori_loop(0, CH // UN, body, 0)
    for k in range(1, NBUF):
        wait_s(k % NBUF)
    plsc.subcore_barrier()
    orows = NPAD // NS
    pltpu.sync_copy(agg.at[pl.ds(s * orows, orows)],
                    out_hbm.at[c, pl.ds(s * orows, orows)])


@functools.cache
def _get_sc_agg():
    return pl.kernel(
        _sc_body,
        out_type=jax.ShapeDtypeStruct((NC, NPAD, D), jnp.float32),
        mesh=plsc.VectorSubcoreMesh(core_axis_name="c", subcore_axis_name="s",
                                    num_cores=NC, num_subcores=NS),
        scratch_types=(
            [pltpu.VMEM((LANE,), jnp.int32) for _ in range(RING)]
            + [pltpu.SemaphoreType.DMA for _ in range(RING)]
            + [pltpu.VMEM((LANE,), jnp.int32) for _ in range(RINGD)]
            + [pltpu.SemaphoreType.DMA for _ in range(RINGD)]
            + [pltpu.VMEM((LANE, D), jnp.float32) for _ in range(NBUF)]
            + [pltpu.SemaphoreType.DMA for _ in range(2 * NBUF)]
            + [pltpu.VMEM_SHARED((NPAD, D), jnp.float32)]
        ),
    )


def kernel(feat, edge_index, etypes,
           W0, bias0, loopW0, ln_g0, ln_b0,
           W1, bias1, loopW1, ln_g1, ln_b1,
           W2, bias2, loopW2, ln_g2, ln_b2):
    src = edge_index[0]
    dst = edge_index[1]
    flat = src * R + etypes
    nw = NC * NS
    rl = E // nw
    padn = CH * LANE - rl
    karr = jnp.arange(padn, dtype=jnp.int32)
    warr = jnp.arange(nw, dtype=jnp.int32).reshape(nw, 1)
    jpt = (NPAD - N) // NS
    idx_pad = (karr * 331 + warr * 77) % (N * R)
    dst_pad = N + (warr % NS) * jpt + karr % jpt
    flat_p = jnp.concatenate([flat.reshape(nw, rl), idx_pad], axis=1)
    dst_p = jnp.concatenate([dst.reshape(nw, rl), dst_pad], axis=1)
    idx3 = flat_p.reshape(NC, NS, CH, LANE)
    dst3 = dst_p.reshape(NC, NS, CH, LANE)
    zeros = jnp.zeros((NPAD // NS, D), jnp.float32)

    sc = _get_sc_agg()
    r1 = lambda v: v.reshape(1, D)
    hw, loop = _pre(feat, W0, loopW0)
    parts = sc(hw.reshape(N * R, D), idx3, dst3, zeros)
    args0 = (parts, loop, r1(ln_g0), r1(ln_b0), r1(bias0))
    hw = _bound_hw(*args0, W1)
    loop = _bound_loop(*args0, loopW1)
    parts = sc(hw.reshape(N * R, D), idx3, dst3, zeros)
    args1 = (parts, loop, r1(ln_g1), r1(ln_b1), r1(bias1))
    hw = _bound_hw(*args1, W2)
    loop = _bound_loop(*args1, loopW2)
    parts = sc(hw.reshape(N * R, D), idx3, dst3, zeros)
    return _post_noact(parts, loop, r1(ln_g2), r1(ln_b2), r1(bias2))
